# baseline (speedup 1.0000x reference)
import functools

import jax
import jax.numpy as jnp
from jax.experimental import pallas as pl
from jax.experimental.pallas import tpu as pltpu


def _ca_body(x_ref, w1_ref, w2_ref, o_ref, *, inv_hw):
    x = x_ref[...].astype(jnp.float32)
    avg = jnp.sum(x, axis=1) * inv_hw
    mx = jnp.max(x, axis=1)
    w1 = w1_ref[...]
    w2 = w2_ref[...]
    h_avg = jnp.maximum(jnp.dot(avg, w1, preferred_element_type=jnp.float32), 0.0)
    h_max = jnp.maximum(jnp.dot(mx, w1, preferred_element_type=jnp.float32), 0.0)
    logits = (jnp.dot(h_avg, w2, preferred_element_type=jnp.float32)
              + jnp.dot(h_max, w2, preferred_element_type=jnp.float32))
    o_ref[:, 0, :] = jax.nn.sigmoid(logits).astype(o_ref.dtype)


def _pick_nb(n, c, hw, itemsize, budget_bytes):
    for cand in (8, 4, 2):
        if n % cand == 0 and n // cand >= 2 and cand * c * hw * itemsize <= budget_bytes:
            return cand
    return 1


def kernel(x, fc1_weight, fc2_weight):
    N, C, H, W = x.shape
    HW = H * W
    Cr = fc1_weight.shape[0]

    x_nhwc = jnp.transpose(x, (0, 2, 3, 1)).reshape(N, HW, C)

    w1 = fc1_weight.reshape(Cr, C).T.astype(jnp.float32)
    w2 = fc2_weight.reshape(C, Cr).T.astype(jnp.float32)

    itemsize = jnp.dtype(x.dtype).itemsize
    nb = _pick_nb(N, C, HW, itemsize, budget_bytes=24 << 20)

    out3d = pl.pallas_call(
        functools.partial(_ca_body, inv_hw=1.0 / float(HW)),
        out_shape=jax.ShapeDtypeStruct((N, 1, C), x.dtype),
        grid=(N // nb,),
        in_specs=[
            pl.BlockSpec((nb, HW, C), lambda n: (n, 0, 0)),
            pl.BlockSpec((C, Cr), lambda n: (0, 0)),
            pl.BlockSpec((Cr, C), lambda n: (0, 0)),
        ],
        out_specs=pl.BlockSpec((nb, 1, C), lambda n: (n, 0, 0)),
        compiler_params=pltpu.CompilerParams(
            dimension_semantics=("parallel",),
            vmem_limit_bytes=56 << 20),
    )(x_nhwc, w1, w2)

    return out3d.reshape(N, C, 1, 1)


# speedup vs baseline: 1.0246x; 1.0246x over previous
import functools

import jax
import jax.numpy as jnp
from jax import lax
from jax.experimental import pallas as pl
from jax.experimental.pallas import tpu as pltpu


def _ca_body(x_ref, w_ref, o_ref, *, inv_hw, nb, cr):
    x = x_ref[...].astype(jnp.float32)
    avg = jnp.sum(x, axis=1) * inv_hw
    mx = jnp.max(x, axis=1)
    both = jnp.concatenate([avg, mx], axis=0)
    w1 = w_ref[:, :cr]
    w2t = w_ref[:, cr:]
    h = jnp.maximum(jnp.dot(both, w1, preferred_element_type=jnp.float32), 0.0)
    hsum = h[:nb] + h[nb:]
    logits = lax.dot_general(hsum, w2t, (((1,), (1,)), ((), ())),
                             preferred_element_type=jnp.float32)
    o_ref[:, 0, :] = jax.nn.sigmoid(logits).astype(o_ref.dtype)


def _pick_nb(n, c, hw, itemsize, budget_bytes):
    for cand in (8, 4, 2):
        if n % cand == 0 and n // cand >= 2 and cand * c * hw * itemsize <= budget_bytes:
            return cand
    return 1


def kernel(x, fc1_weight, fc2_weight):
    N, C, H, W = x.shape
    HW = H * W
    Cr = fc1_weight.shape[0]

    x_nhwc = jnp.transpose(x, (0, 2, 3, 1)).reshape(N, HW, C)

    w_all = jnp.concatenate(
        [fc1_weight.reshape(Cr, C).T, fc2_weight.reshape(C, Cr)],
        axis=1).astype(jnp.float32)

    itemsize = jnp.dtype(x.dtype).itemsize
    nb = _pick_nb(N, C, HW, itemsize, budget_bytes=12 << 20)

    out3d = pl.pallas_call(
        functools.partial(_ca_body, inv_hw=1.0 / float(HW), nb=nb, cr=Cr),
        out_shape=jax.ShapeDtypeStruct((N, 1, C), x.dtype),
        grid=(N // nb,),
        in_specs=[
            pl.BlockSpec((nb, HW, C), lambda n: (n, 0, 0)),
            pl.BlockSpec((C, 2 * Cr), lambda n: (0, 0)),
        ],
        out_specs=pl.BlockSpec((nb, 1, C), lambda n: (n, 0, 0)),
        compiler_params=pltpu.CompilerParams(
            dimension_semantics=("parallel",),
            vmem_limit_bytes=56 << 20),
    )(x_nhwc, w_all)

    return out3d.reshape(N, C, 1, 1)
